# baseline (speedup 1.0000x reference)
import jax
import jax.numpy as jnp
from jax import lax
from jax.experimental import pallas as pl
from jax.experimental.pallas import tpu as pltpu
from jax.experimental.pallas import tpu_sc as plsc

H = 224
NB = 1536
NW = 32
BPW = NB // NW


def _sc_body(x_hbm, idx_hbm, out_hbm, idx_v, shared, g0, g1, s0, s1):
    sid = lax.axis_index("s")
    wid = sid * 2 + lax.axis_index("c")
    base = wid * BPW
    pltpu.sync_copy(idx_hbm.at[pl.ds(base, BPW)], idx_v)

    gsems = (g0, g1)
    ssems = (s0, s1)

    def src_row(i):
        return idx_v[pl.ds((i // 16) * 16, 16)][i % 16]

    def buf(i):
        return shared.at[sid, i % 2]

    def start_gather(i):
        pltpu.async_copy(x_hbm.at[pl.ds(src_row(i), 1)], buf(i), gsems[i % 2])

    def wait_gather(i):
        pltpu.make_async_copy(
            x_hbm.at[pl.ds(0, 1)], buf(i), gsems[i % 2]
        ).wait()

    def start_store(i):
        pltpu.async_copy(buf(i), out_hbm.at[pl.ds(base + i, 1)], ssems[i % 2])

    def wait_store(i):
        pltpu.make_async_copy(
            buf(i), out_hbm.at[pl.ds(base + i, 1)], ssems[i % 2]
        ).wait()

    start_gather(0)
    start_gather(1)
    for i in range(BPW):
        wait_gather(i)
        start_store(i)
        if i + 2 < BPW:
            wait_store(i)
            start_gather(i + 2)
    wait_store(BPW - 2)
    wait_store(BPW - 1)


def kernel(x, permutation):
    b, c, h, w = x.shape
    xr = x.reshape(NB, H, H)
    idx = (
        jnp.arange(b, dtype=jnp.int32)[:, None] * c
        + permutation.astype(jnp.int32)[None, :]
    ).reshape(NB)
    mesh = plsc.VectorSubcoreMesh(core_axis_name="c", subcore_axis_name="s")
    out = pl.kernel(
        _sc_body,
        mesh=mesh,
        out_type=jax.ShapeDtypeStruct((NB, H, H), x.dtype),
        scratch_types=[
            pltpu.VMEM((BPW,), jnp.int32),
            pltpu.VMEM_SHARED((16, 2, 1, H, H), jnp.float32),
            pltpu.SemaphoreType.DMA,
            pltpu.SemaphoreType.DMA,
            pltpu.SemaphoreType.DMA,
            pltpu.SemaphoreType.DMA,
        ],
    )(xr, idx)
    return out.reshape(b, c, h, w)
